# Initial kernel scaffold; baseline (speedup 1.0000x reference)
import jax
import jax.numpy as jnp
from jax.experimental import pallas as pl
from jax.experimental.pallas import tpu as pltpu


def _body(x_ref, w_ref, b_ref, o_ref, xp_ref, xcat_ref):
    xp_ref[0:1] = jnp.zeros((1, 66, 256), jnp.float32)
    xp_ref[65:66] = jnp.zeros((1, 66, 256), jnp.float32)
    xp_ref[1:65, 0:1] = jnp.zeros((64, 1, 256), jnp.float32)
    xp_ref[1:65, 65:66] = jnp.zeros((64, 1, 256), jnp.float32)
    xp_ref[1:65, 1:65] = x_ref[0]
    xp = xp_ref[...]
    for kh in range(3):
        for kw in range(3):
            j = kh * 3 + kw
            xs = jax.lax.slice(xp, (kh, kw, 0), (kh + 63, kw + 63, 256),
                               (2, 2, 1))
            xcat_ref[:, j * 256:(j + 1) * 256] = xs.reshape(1024, 256)
    acc = jnp.dot(xcat_ref[...], w_ref[...],
                  preferred_element_type=jnp.float32)
    acc = acc + b_ref[0, :][None, :]
    a3 = acc.reshape(32, 32, 256)
    gs = []
    for si in range(2):
        for sj in range(2):
            g = jax.lax.slice(a3, (si, sj, 0), (32, 32, 256), (2, 2, 1))
            gs.append(g.reshape(256, 256))
    perm = jnp.concatenate(gs, axis=0)
    o_ref[0] = perm.T


def kernel(x, weight, bias):
    xt = jnp.transpose(x, (0, 2, 3, 1))
    wc = jnp.transpose(weight, (2, 3, 1, 0)).reshape(9 * 256, 256)
    b2 = bias.reshape(1, 256)
    return pl.pallas_call(
        _body,
        out_shape=jax.ShapeDtypeStruct((16, 256, 1024), jnp.float32),
        grid=(16,),
        in_specs=[
            pl.BlockSpec((1, 64, 64, 256), lambda i: (i, 0, 0, 0)),
            pl.BlockSpec((2304, 256), lambda i: (0, 0)),
            pl.BlockSpec((1, 256), lambda i: (0, 0)),
        ],
        out_specs=pl.BlockSpec((1, 256, 1024), lambda i: (i, 0, 0)),
        scratch_shapes=[
            pltpu.VMEM((66, 66, 256), jnp.float32),
            pltpu.VMEM((1024, 2304), jnp.float32),
        ],
        compiler_params=pltpu.CompilerParams(
            dimension_semantics=("parallel",),
            vmem_limit_bytes=52 * 1024 * 1024,
        ),
        name="conv2d_orion",
    )(xt, wc, b2)


# speedup vs baseline: 2.5125x; 2.5125x over previous
import jax
import jax.numpy as jnp
from jax.experimental import pallas as pl
from jax.experimental.pallas import tpu as pltpu


def _body(x_ref, w_ref, b_ref, o_ref, xp0_ref, xp1_ref, xcat_ref):
    for t, xpt in enumerate((xp0_ref, xp1_ref)):
        xpt[0:1] = jnp.zeros((1, 66, 128), jnp.float32)
        xpt[65:66] = jnp.zeros((1, 66, 128), jnp.float32)
        xpt[1:65, 0:1] = jnp.zeros((64, 1, 128), jnp.float32)
        xpt[1:65, 65:66] = jnp.zeros((64, 1, 128), jnp.float32)
        xpt[1:65, 1:65] = x_ref[0, :, :, t * 128:(t + 1) * 128]
    for g, (si, sj) in enumerate(((0, 0), (0, 1), (1, 0), (1, 1))):
        for kh in range(3):
            for kw in range(3):
                j = kh * 3 + kw
                for t, xpt in enumerate((xp0_ref, xp1_ref)):
                    xs = xpt[pl.ds(2 * si + kh, 16, 4),
                             pl.ds(2 * sj + kw, 16, 4), :]
                    xcat_ref[g * 256:(g + 1) * 256,
                             j * 256 + t * 128:j * 256 + (t + 1) * 128] = (
                        xs.reshape(256, 128))
    acc = jnp.dot(xcat_ref[...], w_ref[...],
                  preferred_element_type=jnp.float32)
    acc = acc + b_ref[0, :][None, :]
    o_ref[0] = acc.T


def kernel(x, weight, bias):
    xt = jnp.transpose(x, (0, 2, 3, 1))
    wc = jnp.transpose(weight, (2, 3, 1, 0)).reshape(9 * 256, 256)
    b2 = bias.reshape(1, 256)
    return pl.pallas_call(
        _body,
        out_shape=jax.ShapeDtypeStruct((16, 256, 1024), jnp.float32),
        grid=(16,),
        in_specs=[
            pl.BlockSpec((1, 64, 64, 256), lambda i: (i, 0, 0, 0)),
            pl.BlockSpec((2304, 256), lambda i: (0, 0)),
            pl.BlockSpec((1, 256), lambda i: (0, 0)),
        ],
        out_specs=pl.BlockSpec((1, 256, 1024), lambda i: (i, 0, 0)),
        scratch_shapes=[
            pltpu.VMEM((66, 66, 128), jnp.float32),
            pltpu.VMEM((66, 66, 128), jnp.float32),
            pltpu.VMEM((1024, 2304), jnp.float32),
        ],
        compiler_params=pltpu.CompilerParams(
            dimension_semantics=("parallel",),
            vmem_limit_bytes=52 * 1024 * 1024,
        ),
        name="conv2d_orion",
    )(xt, wc, b2)


# speedup vs baseline: 2.5376x; 1.0100x over previous
import jax
import jax.numpy as jnp
from jax.experimental import pallas as pl
from jax.experimental.pallas import tpu as pltpu


def _body(x_ref, w_ref, b_ref, o_ref, xp0_ref, xp1_ref, xcat_ref):
    for t, xpt in enumerate((xp0_ref, xp1_ref)):
        xpt[0:1] = jnp.zeros((1, 66, 128), jnp.float32)
        xpt[65:66] = jnp.zeros((1, 66, 128), jnp.float32)
        xpt[1:65, 0:1] = jnp.zeros((64, 1, 128), jnp.float32)
        xpt[1:65, 65:66] = jnp.zeros((64, 1, 128), jnp.float32)
        xpt[1:65, 1:65] = x_ref[0, :, :, t * 128:(t + 1) * 128]
    for g, (si, sj) in enumerate(((0, 0), (0, 1), (1, 0), (1, 1))):
        for kh in range(3):
            for kw in range(3):
                j = kh * 3 + kw
                for t, xpt in enumerate((xp0_ref, xp1_ref)):
                    xs = xpt[pl.ds(2 * si + kh, 16, 4),
                             pl.ds(2 * sj + kw, 16, 4), :]
                    xcat_ref[g * 256:(g + 1) * 256,
                             j * 256 + t * 128:j * 256 + (t + 1) * 128] = (
                        xs.reshape(256, 128).astype(jnp.bfloat16))
    acc = jnp.dot(xcat_ref[...], w_ref[...],
                  preferred_element_type=jnp.float32)
    acc = acc + b_ref[0, :][None, :]
    o_ref[0] = acc.T


def kernel(x, weight, bias):
    xt = jnp.transpose(x, (0, 2, 3, 1))
    wc = jnp.transpose(weight, (2, 3, 1, 0)).reshape(2304, 256)
    wc = wc.astype(jnp.bfloat16)
    b2 = bias.reshape(1, 256)
    return pl.pallas_call(
        _body,
        out_shape=jax.ShapeDtypeStruct((16, 256, 1024), jnp.float32),
        grid=(16,),
        in_specs=[
            pl.BlockSpec((1, 64, 64, 256), lambda i: (i, 0, 0, 0)),
            pl.BlockSpec((2304, 256), lambda i: (0, 0)),
            pl.BlockSpec((1, 256), lambda i: (0, 0)),
        ],
        out_specs=pl.BlockSpec((1, 256, 1024), lambda i: (i, 0, 0)),
        scratch_shapes=[
            pltpu.VMEM((66, 66, 128), jnp.float32),
            pltpu.VMEM((66, 66, 128), jnp.float32),
            pltpu.VMEM((1024, 2304), jnp.bfloat16),
        ],
        compiler_params=pltpu.CompilerParams(
            dimension_semantics=("parallel",),
            vmem_limit_bytes=52 * 1024 * 1024,
        ),
        name="conv2d_orion",
    )(xt, wc, b2)
